# Initial kernel scaffold; baseline (speedup 1.0000x reference)
import functools

import jax
import jax.numpy as jnp
from jax import lax
from jax.experimental import pallas as pl
from jax.experimental.pallas import tpu as pltpu
from jax.experimental.pallas import tpu_sc as plsc

N = 10000
E = 320000
D = 16
NPAD = 10240
NC, NS = 2, 16
NW = NC * NS
CH = 128
EPT = 10240
NCHUNK = EPT // CH
EPAD = NW * EPT
ROWS_PER_TILE = NPAD // NS


def _sc_propagate_body(table, src3, dst3, zeros, out, src_v, dst_v, rows_v,
                       sem, acc):
    c = lax.axis_index("c")
    s = lax.axis_index("s")

    pltpu.sync_copy(zeros.at[pl.ds(s * ROWS_PER_TILE, ROWS_PER_TILE)],
                    acc.at[pl.ds(s * ROWS_PER_TILE, ROWS_PER_TILE)])

    w = c * NS + s
    pltpu.sync_copy(src3.at[w], src_v)
    pltpu.sync_copy(dst3.at[w], dst_v)

    plsc.subcore_barrier()

    def chunk(j, carry):
        pltpu.async_copy(table.at[src_v.at[j]], rows_v, sem).wait()
        pltpu.sync_copy(rows_v, acc.at[dst_v.at[j]], add=True)
        return carry

    lax.fori_loop(0, NCHUNK, chunk, 0)

    plsc.subcore_barrier()

    pltpu.sync_copy(acc.at[pl.ds(s * ROWS_PER_TILE, ROWS_PER_TILE)],
                    out.at[c, pl.ds(s * ROWS_PER_TILE, ROWS_PER_TILE)])


_sc_propagate = pl.kernel(
    _sc_propagate_body,
    out_type=jax.ShapeDtypeStruct((NC, NPAD, D), jnp.float32),
    mesh=plsc.VectorSubcoreMesh(core_axis_name="c", subcore_axis_name="s"),
    scratch_types=[
        pltpu.VMEM((NCHUNK, CH), jnp.int32),
        pltpu.VMEM((NCHUNK, CH), jnp.int32),
        pltpu.VMEM((CH, D), jnp.float32),
        pltpu.SemaphoreType.DMA,
        pltpu.VMEM_SHARED((NPAD, D), jnp.float32),
    ],
)


_RB = 1280
_GRID = NPAD // _RB


def _pre_body(x_ref, w_ref, d0_ref, d1_ref, t_ref, dinv_ref):
    deg = d0_ref[...] + d1_ref[...] + 1.0
    dinv = lax.rsqrt(deg)
    h = jnp.dot(x_ref[...], w_ref[...], preferred_element_type=jnp.float32)
    dinv_ref[...] = dinv
    t_ref[...] = h * dinv


def _pre(x, W1, d0, d1):
    return pl.pallas_call(
        _pre_body,
        grid=(_GRID,),
        in_specs=[
            pl.BlockSpec((_RB, 128), lambda i: (i, 0)),
            pl.BlockSpec((128, D), lambda i: (0, 0)),
            pl.BlockSpec((_RB, D), lambda i: (i, 0)),
            pl.BlockSpec((_RB, D), lambda i: (i, 0)),
        ],
        out_specs=[
            pl.BlockSpec((_RB, D), lambda i: (i, 0)),
            pl.BlockSpec((_RB, D), lambda i: (i, 0)),
        ],
        out_shape=[
            jax.ShapeDtypeStruct((NPAD, D), jnp.float32),
            jax.ShapeDtypeStruct((NPAD, D), jnp.float32),
        ],
    )(x, W1, d0, d1)


def _bnd1_body(p_ref, t_ref, dinv_ref, b_ref, o_ref):
    q = dinv_ref[...] * (p_ref[0] + p_ref[1] + t_ref[...])
    a = jnp.maximum(q + b_ref[...], 0.0)
    o_ref[...] = dinv_ref[...] * a


def _boundary1(p, t, dinv, b):
    return pl.pallas_call(
        _bnd1_body,
        grid=(_GRID,),
        in_specs=[
            pl.BlockSpec((NC, _RB, D), lambda i: (0, i, 0)),
            pl.BlockSpec((_RB, D), lambda i: (i, 0)),
            pl.BlockSpec((_RB, D), lambda i: (i, 0)),
            pl.BlockSpec((1, D), lambda i: (0, 0)),
        ],
        out_specs=pl.BlockSpec((_RB, D), lambda i: (i, 0)),
        out_shape=jax.ShapeDtypeStruct((NPAD, D), jnp.float32),
    )(p, t, dinv, b.reshape(1, D))


def _bnd_body(p_ref, t_ref, dinv_ref, w_ref, b_ref, o_ref):
    q = dinv_ref[...] * (p_ref[0] + p_ref[1] + t_ref[...])
    z = jnp.dot(q, w_ref[...], preferred_element_type=jnp.float32)
    a = jnp.maximum(z + b_ref[...], 0.0)
    o_ref[...] = dinv_ref[...] * a


def _boundary(p, t, dinv, W, b):
    return pl.pallas_call(
        _bnd_body,
        grid=(_GRID,),
        in_specs=[
            pl.BlockSpec((NC, _RB, D), lambda i: (0, i, 0)),
            pl.BlockSpec((_RB, D), lambda i: (i, 0)),
            pl.BlockSpec((_RB, D), lambda i: (i, 0)),
            pl.BlockSpec((D, D), lambda i: (0, 0)),
            pl.BlockSpec((1, D), lambda i: (0, 0)),
        ],
        out_specs=pl.BlockSpec((_RB, D), lambda i: (i, 0)),
        out_shape=jax.ShapeDtypeStruct((NPAD, D), jnp.float32),
    )(p, t, dinv, W, b.reshape(1, D))


def _final_body(p_ref, t_ref, dinv_ref, w_ref, b_ref, o_ref):
    q = dinv_ref[...] * (p_ref[0] + p_ref[1] + t_ref[...])
    z = jnp.dot(q, w_ref[...], preferred_element_type=jnp.float32)
    z = z + b_ref[...]
    m = jnp.max(z, axis=1, keepdims=True)
    zs = z - m
    lse = jnp.log(jnp.sum(jnp.exp(zs), axis=1, keepdims=True))
    o_ref[...] = zs - lse


def _final(p, t, dinv, W, b, dout):
    return pl.pallas_call(
        _final_body,
        grid=(_GRID,),
        in_specs=[
            pl.BlockSpec((NC, _RB, D), lambda i: (0, i, 0)),
            pl.BlockSpec((_RB, D), lambda i: (i, 0)),
            pl.BlockSpec((_RB, D), lambda i: (i, 0)),
            pl.BlockSpec((D, dout), lambda i: (0, 0)),
            pl.BlockSpec((1, dout), lambda i: (0, 0)),
        ],
        out_specs=pl.BlockSpec((_RB, dout), lambda i: (i, 0)),
        out_shape=jax.ShapeDtypeStruct((NPAD, dout), jnp.float32),
    )(p, t, dinv, W, b.reshape(1, dout))


@jax.jit
def _run(x, edge_index, W1, b1, W2, b2, W3, b3, W4, b4, W5, b5, W6, b6):
    src = edge_index[0].astype(jnp.int32)
    dst = edge_index[1].astype(jnp.int32)
    fill = jnp.full((EPAD - E,), NPAD - 1, dtype=jnp.int32)
    src3 = jnp.concatenate([src, fill]).reshape(NW, NCHUNK, CH)
    dst3 = jnp.concatenate([dst, fill]).reshape(NW, NCHUNK, CH)

    zeros_t = jnp.zeros((NPAD, D), jnp.float32)
    ones_t = jnp.concatenate(
        [jnp.ones((N, D), jnp.float32), jnp.zeros((NPAD - N, D), jnp.float32)])
    x_pad = jnp.concatenate([x, jnp.zeros((NPAD - N, 128), jnp.float32)])

    degp = _sc_propagate(ones_t, src3, dst3, zeros_t)

    t, dinv = _pre(x_pad, W1, degp[0], degp[1])

    p = _sc_propagate(t, src3, dst3, zeros_t)
    t = _boundary1(p, t, dinv, b1)

    for W, b in ((W2, b2), (W3, b3), (W4, b4), (W5, b5)):
        p = _sc_propagate(t, src3, dst3, zeros_t)
        t = _boundary(p, t, dinv, W, b)

    p = _sc_propagate(t, src3, dst3, zeros_t)
    out = _final(p, t, dinv, W6, b6, W6.shape[1])
    return out[:N]


def kernel(x, edge_index, W1, b1, W2, b2, W3, b3, W4, b4, W5, b5, W6, b6):
    return _run(x, edge_index, W1, b1, W2, b2, W3, b3, W4, b4, W5, b5, W6, b6)


# speedup vs baseline: 19.6487x; 19.6487x over previous
import functools

import jax
import jax.numpy as jnp
from jax import lax
from jax.experimental import pallas as pl
from jax.experimental.pallas import tpu as pltpu
from jax.experimental.pallas import tpu_sc as plsc

N = 10000
E = 320000
D = 16
NPAD = 10240
NC, NS = 2, 16
NW = NC * NS
CH = 128
EPT = 10240
NCHUNK = EPT // CH
EPAD = NW * EPT
ROWS_PER_TILE = NPAD // NS


def _sc_propagate_body(table, src3, dst3, zeros, out, src_v, dst_v, rows_v,
                       sem, acc):
    c = lax.axis_index("c")
    s = lax.axis_index("s")

    pltpu.sync_copy(zeros.at[pl.ds(s * ROWS_PER_TILE, ROWS_PER_TILE)],
                    acc.at[pl.ds(s * ROWS_PER_TILE, ROWS_PER_TILE)])

    w = c * NS + s
    pltpu.sync_copy(src3.at[w], src_v)
    pltpu.sync_copy(dst3.at[w], dst_v)

    plsc.subcore_barrier()

    def chunk(j, carry):
        pltpu.async_copy(table.at[src_v.at[j]], rows_v, sem).wait()
        pltpu.sync_copy(rows_v, acc.at[dst_v.at[j]], add=True)
        return carry

    lax.fori_loop(0, NCHUNK, chunk, 0)

    plsc.subcore_barrier()

    pltpu.sync_copy(acc.at[pl.ds(s * ROWS_PER_TILE, ROWS_PER_TILE)],
                    out.at[c, pl.ds(s * ROWS_PER_TILE, ROWS_PER_TILE)])


_sc_propagate = pl.kernel(
    _sc_propagate_body,
    out_type=jax.ShapeDtypeStruct((NC, NPAD, D), jnp.float32),
    mesh=plsc.VectorSubcoreMesh(core_axis_name="c", subcore_axis_name="s"),
    scratch_types=[
        pltpu.VMEM((NCHUNK, CH), jnp.int32),
        pltpu.VMEM((NCHUNK, CH), jnp.int32),
        pltpu.VMEM((CH, D), jnp.float32),
        pltpu.SemaphoreType.DMA,
        pltpu.VMEM_SHARED((NPAD, D), jnp.float32),
    ],
    compiler_params=pltpu.CompilerParams(use_tc_tiling_on_sc=False),
)


_RB = 1280
_GRID = NPAD // _RB


def _pre_body(x_ref, w_ref, d0_ref, d1_ref, t_ref, dinv_ref):
    deg = d0_ref[...] + d1_ref[...] + 1.0
    dinv = lax.rsqrt(deg)
    h = jnp.dot(x_ref[...], w_ref[...], preferred_element_type=jnp.float32)
    dinv_ref[...] = dinv
    t_ref[...] = h * dinv


def _pre(x, W1, d0, d1):
    return pl.pallas_call(
        _pre_body,
        grid=(_GRID,),
        in_specs=[
            pl.BlockSpec((_RB, 128), lambda i: (i, 0)),
            pl.BlockSpec((128, D), lambda i: (0, 0)),
            pl.BlockSpec((_RB, D), lambda i: (i, 0)),
            pl.BlockSpec((_RB, D), lambda i: (i, 0)),
        ],
        out_specs=[
            pl.BlockSpec((_RB, D), lambda i: (i, 0)),
            pl.BlockSpec((_RB, D), lambda i: (i, 0)),
        ],
        out_shape=[
            jax.ShapeDtypeStruct((NPAD, D), jnp.float32),
            jax.ShapeDtypeStruct((NPAD, D), jnp.float32),
        ],
    )(x, W1, d0, d1)


def _bnd1_body(p_ref, t_ref, dinv_ref, b_ref, o_ref):
    q = dinv_ref[...] * (p_ref[0] + p_ref[1] + t_ref[...])
    a = jnp.maximum(q + b_ref[...], 0.0)
    o_ref[...] = dinv_ref[...] * a


def _boundary1(p, t, dinv, b):
    return pl.pallas_call(
        _bnd1_body,
        grid=(_GRID,),
        in_specs=[
            pl.BlockSpec((NC, _RB, D), lambda i: (0, i, 0)),
            pl.BlockSpec((_RB, D), lambda i: (i, 0)),
            pl.BlockSpec((_RB, D), lambda i: (i, 0)),
            pl.BlockSpec((1, D), lambda i: (0, 0)),
        ],
        out_specs=pl.BlockSpec((_RB, D), lambda i: (i, 0)),
        out_shape=jax.ShapeDtypeStruct((NPAD, D), jnp.float32),
    )(p, t, dinv, b.reshape(1, D))


def _bnd_body(p_ref, t_ref, dinv_ref, w_ref, b_ref, o_ref):
    q = dinv_ref[...] * (p_ref[0] + p_ref[1] + t_ref[...])
    z = jnp.dot(q, w_ref[...], preferred_element_type=jnp.float32)
    a = jnp.maximum(z + b_ref[...], 0.0)
    o_ref[...] = dinv_ref[...] * a


def _boundary(p, t, dinv, W, b):
    return pl.pallas_call(
        _bnd_body,
        grid=(_GRID,),
        in_specs=[
            pl.BlockSpec((NC, _RB, D), lambda i: (0, i, 0)),
            pl.BlockSpec((_RB, D), lambda i: (i, 0)),
            pl.BlockSpec((_RB, D), lambda i: (i, 0)),
            pl.BlockSpec((D, D), lambda i: (0, 0)),
            pl.BlockSpec((1, D), lambda i: (0, 0)),
        ],
        out_specs=pl.BlockSpec((_RB, D), lambda i: (i, 0)),
        out_shape=jax.ShapeDtypeStruct((NPAD, D), jnp.float32),
    )(p, t, dinv, W, b.reshape(1, D))


def _final_body(p_ref, t_ref, dinv_ref, w_ref, b_ref, o_ref):
    q = dinv_ref[...] * (p_ref[0] + p_ref[1] + t_ref[...])
    z = jnp.dot(q, w_ref[...], preferred_element_type=jnp.float32)
    z = z + b_ref[...]
    m = jnp.max(z, axis=1, keepdims=True)
    zs = z - m
    lse = jnp.log(jnp.sum(jnp.exp(zs), axis=1, keepdims=True))
    o_ref[...] = zs - lse


def _final(p, t, dinv, W, b, dout):
    return pl.pallas_call(
        _final_body,
        grid=(_GRID,),
        in_specs=[
            pl.BlockSpec((NC, _RB, D), lambda i: (0, i, 0)),
            pl.BlockSpec((_RB, D), lambda i: (i, 0)),
            pl.BlockSpec((_RB, D), lambda i: (i, 0)),
            pl.BlockSpec((D, dout), lambda i: (0, 0)),
            pl.BlockSpec((1, dout), lambda i: (0, 0)),
        ],
        out_specs=pl.BlockSpec((_RB, dout), lambda i: (i, 0)),
        out_shape=jax.ShapeDtypeStruct((NPAD, dout), jnp.float32),
    )(p, t, dinv, W, b.reshape(1, dout))


@jax.jit
def _run(x, edge_index, W1, b1, W2, b2, W3, b3, W4, b4, W5, b5, W6, b6):
    src = edge_index[0].astype(jnp.int32)
    dst = edge_index[1].astype(jnp.int32)
    fill = jnp.full((EPAD - E,), NPAD - 1, dtype=jnp.int32)
    src3 = jnp.concatenate([src, fill]).reshape(NW, NCHUNK, CH)
    dst3 = jnp.concatenate([dst, fill]).reshape(NW, NCHUNK, CH)

    zeros_t = jnp.zeros((NPAD, D), jnp.float32)
    ones_t = jnp.concatenate(
        [jnp.ones((N, D), jnp.float32), jnp.zeros((NPAD - N, D), jnp.float32)])
    x_pad = jnp.concatenate([x, jnp.zeros((NPAD - N, 128), jnp.float32)])

    degp = _sc_propagate(ones_t, src3, dst3, zeros_t)

    t, dinv = _pre(x_pad, W1, degp[0], degp[1])

    p = _sc_propagate(t, src3, dst3, zeros_t)
    t = _boundary1(p, t, dinv, b1)

    for W, b in ((W2, b2), (W3, b3), (W4, b4), (W5, b5)):
        p = _sc_propagate(t, src3, dst3, zeros_t)
        t = _boundary(p, t, dinv, W, b)

    p = _sc_propagate(t, src3, dst3, zeros_t)
    out = _final(p, t, dinv, W6, b6, W6.shape[1])
    return out[:N]


def kernel(x, edge_index, W1, b1, W2, b2, W3, b3, W4, b4, W5, b5, W6, b6):
    return _run(x, edge_index, W1, b1, W2, b2, W3, b3, W4, b4, W5, b5, W6, b6)


# speedup vs baseline: 30.0435x; 1.5290x over previous
import functools

import jax
import jax.numpy as jnp
from jax import lax
from jax.experimental import pallas as pl
from jax.experimental.pallas import tpu as pltpu
from jax.experimental.pallas import tpu_sc as plsc

N = 10000
E = 320000
D = 16
NPAD = 10240
NC, NS = 2, 16
NW = NC * NS
CH = 128
EPT = 10240
NCHUNK = EPT // CH
EPAD = NW * EPT
ROWS_PER_TILE = NPAD // NS


def _sc_propagate_body(table, src3, dst3, zeros, out, src_v, dst_v, rows_v,
                       sems, acc):
    c = lax.axis_index("c")
    s = lax.axis_index("s")

    pltpu.sync_copy(zeros.at[pl.ds(s * ROWS_PER_TILE, ROWS_PER_TILE)],
                    acc.at[pl.ds(s * ROWS_PER_TILE, ROWS_PER_TILE)])

    w = c * NS + s
    pltpu.sync_copy(src3.at[w], src_v)
    pltpu.sync_copy(dst3.at[w], dst_v)

    plsc.subcore_barrier()

    pltpu.async_copy(table.at[src_v.at[0]], rows_v.at[0], sems.at[0])

    def chunk(j, carry):
        nxt = j + 1

        @pl.when(nxt < NCHUNK)
        def _():
            pltpu.async_copy(table.at[src_v.at[nxt]], rows_v.at[nxt % 2],
                             sems.at[nxt % 2])

        pltpu.make_async_copy(table.at[src_v.at[j]], rows_v.at[j % 2],
                              sems.at[j % 2]).wait()
        pltpu.sync_copy(rows_v.at[j % 2], acc.at[dst_v.at[j]], add=True)
        return carry

    lax.fori_loop(0, NCHUNK, chunk, 0)

    plsc.subcore_barrier()

    pltpu.sync_copy(acc.at[pl.ds(s * ROWS_PER_TILE, ROWS_PER_TILE)],
                    out.at[c, pl.ds(s * ROWS_PER_TILE, ROWS_PER_TILE)])


_sc_propagate = pl.kernel(
    _sc_propagate_body,
    out_type=jax.ShapeDtypeStruct((NC, NPAD, D), jnp.float32),
    mesh=plsc.VectorSubcoreMesh(core_axis_name="c", subcore_axis_name="s"),
    scratch_types=[
        pltpu.VMEM((NCHUNK, CH), jnp.int32),
        pltpu.VMEM((NCHUNK, CH), jnp.int32),
        pltpu.VMEM((2, CH, D), jnp.float32),
        pltpu.SemaphoreType.DMA((2,)),
        pltpu.VMEM_SHARED((NPAD, D), jnp.float32),
    ],
    compiler_params=pltpu.CompilerParams(use_tc_tiling_on_sc=False),
)


def _sc_degree_body(dst3, zeros, ones16, out, dst_v, ones_v, acc):
    c = lax.axis_index("c")
    s = lax.axis_index("s")

    pltpu.sync_copy(zeros.at[pl.ds(s * ROWS_PER_TILE, ROWS_PER_TILE)],
                    acc.at[pl.ds(s * ROWS_PER_TILE, ROWS_PER_TILE)])

    w = c * NS + s
    pltpu.sync_copy(dst3.at[w], dst_v)
    pltpu.sync_copy(ones16, ones_v)

    plsc.subcore_barrier()

    def chunk(j, carry):
        pltpu.sync_copy(ones_v, acc.at[dst_v.at[j]], add=True)
        return carry

    lax.fori_loop(0, NCHUNK, chunk, 0)

    plsc.subcore_barrier()

    pltpu.sync_copy(acc.at[pl.ds(s * ROWS_PER_TILE, ROWS_PER_TILE)],
                    out.at[c, pl.ds(s * ROWS_PER_TILE, ROWS_PER_TILE)])


_sc_degree = pl.kernel(
    _sc_degree_body,
    out_type=jax.ShapeDtypeStruct((NC, NPAD, D), jnp.float32),
    mesh=plsc.VectorSubcoreMesh(core_axis_name="c", subcore_axis_name="s"),
    scratch_types=[
        pltpu.VMEM((NCHUNK, CH), jnp.int32),
        pltpu.VMEM((CH, D), jnp.float32),
        pltpu.VMEM_SHARED((NPAD, D), jnp.float32),
    ],
    compiler_params=pltpu.CompilerParams(use_tc_tiling_on_sc=False),
)


_RB = 1280
_GRID = NPAD // _RB


def _pre_body(x_ref, w_ref, d0_ref, d1_ref, t_ref, dinv_ref):
    deg = d0_ref[...] + d1_ref[...] + 1.0
    dinv = lax.rsqrt(deg)
    h = jnp.dot(x_ref[...], w_ref[...], preferred_element_type=jnp.float32)
    dinv_ref[...] = dinv
    t_ref[...] = h * dinv


def _pre(x, W1, d0, d1):
    return pl.pallas_call(
        _pre_body,
        grid=(_GRID,),
        in_specs=[
            pl.BlockSpec((_RB, 128), lambda i: (i, 0)),
            pl.BlockSpec((128, D), lambda i: (0, 0)),
            pl.BlockSpec((_RB, D), lambda i: (i, 0)),
            pl.BlockSpec((_RB, D), lambda i: (i, 0)),
        ],
        out_specs=[
            pl.BlockSpec((_RB, D), lambda i: (i, 0)),
            pl.BlockSpec((_RB, D), lambda i: (i, 0)),
        ],
        out_shape=[
            jax.ShapeDtypeStruct((NPAD, D), jnp.float32),
            jax.ShapeDtypeStruct((NPAD, D), jnp.float32),
        ],
    )(x, W1, d0, d1)


def _bnd1_body(p_ref, t_ref, dinv_ref, b_ref, o_ref):
    q = dinv_ref[...] * (p_ref[0] + p_ref[1] + t_ref[...])
    a = jnp.maximum(q + b_ref[...], 0.0)
    o_ref[...] = dinv_ref[...] * a


def _boundary1(p, t, dinv, b):
    return pl.pallas_call(
        _bnd1_body,
        grid=(_GRID,),
        in_specs=[
            pl.BlockSpec((NC, _RB, D), lambda i: (0, i, 0)),
            pl.BlockSpec((_RB, D), lambda i: (i, 0)),
            pl.BlockSpec((_RB, D), lambda i: (i, 0)),
            pl.BlockSpec((1, D), lambda i: (0, 0)),
        ],
        out_specs=pl.BlockSpec((_RB, D), lambda i: (i, 0)),
        out_shape=jax.ShapeDtypeStruct((NPAD, D), jnp.float32),
    )(p, t, dinv, b.reshape(1, D))


def _bnd_body(p_ref, t_ref, dinv_ref, w_ref, b_ref, o_ref):
    q = dinv_ref[...] * (p_ref[0] + p_ref[1] + t_ref[...])
    z = jnp.dot(q, w_ref[...], preferred_element_type=jnp.float32)
    a = jnp.maximum(z + b_ref[...], 0.0)
    o_ref[...] = dinv_ref[...] * a


def _boundary(p, t, dinv, W, b):
    return pl.pallas_call(
        _bnd_body,
        grid=(_GRID,),
        in_specs=[
            pl.BlockSpec((NC, _RB, D), lambda i: (0, i, 0)),
            pl.BlockSpec((_RB, D), lambda i: (i, 0)),
            pl.BlockSpec((_RB, D), lambda i: (i, 0)),
            pl.BlockSpec((D, D), lambda i: (0, 0)),
            pl.BlockSpec((1, D), lambda i: (0, 0)),
        ],
        out_specs=pl.BlockSpec((_RB, D), lambda i: (i, 0)),
        out_shape=jax.ShapeDtypeStruct((NPAD, D), jnp.float32),
    )(p, t, dinv, W, b.reshape(1, D))


def _final_body(p_ref, t_ref, dinv_ref, w_ref, b_ref, o_ref):
    q = dinv_ref[...] * (p_ref[0] + p_ref[1] + t_ref[...])
    z = jnp.dot(q, w_ref[...], preferred_element_type=jnp.float32)
    z = z + b_ref[...]
    m = jnp.max(z, axis=1, keepdims=True)
    zs = z - m
    lse = jnp.log(jnp.sum(jnp.exp(zs), axis=1, keepdims=True))
    o_ref[...] = zs - lse


def _final(p, t, dinv, W, b, dout):
    return pl.pallas_call(
        _final_body,
        grid=(_GRID,),
        in_specs=[
            pl.BlockSpec((NC, _RB, D), lambda i: (0, i, 0)),
            pl.BlockSpec((_RB, D), lambda i: (i, 0)),
            pl.BlockSpec((_RB, D), lambda i: (i, 0)),
            pl.BlockSpec((D, dout), lambda i: (0, 0)),
            pl.BlockSpec((1, dout), lambda i: (0, 0)),
        ],
        out_specs=pl.BlockSpec((_RB, dout), lambda i: (i, 0)),
        out_shape=jax.ShapeDtypeStruct((NPAD, dout), jnp.float32),
    )(p, t, dinv, W, b.reshape(1, dout))


@jax.jit
def _run(x, edge_index, W1, b1, W2, b2, W3, b3, W4, b4, W5, b5, W6, b6):
    src = edge_index[0].astype(jnp.int32)
    dst = edge_index[1].astype(jnp.int32)
    fill = jnp.full((EPAD - E,), NPAD - 1, dtype=jnp.int32)
    src3 = jnp.concatenate([src, fill]).reshape(NW, NCHUNK, CH)
    dst3 = jnp.concatenate([dst, fill]).reshape(NW, NCHUNK, CH)

    zeros_t = jnp.zeros((NPAD, D), jnp.float32)
    ones_c = jnp.ones((CH, D), jnp.float32)
    x_pad = jnp.concatenate([x, jnp.zeros((NPAD - N, 128), jnp.float32)])

    degp = _sc_degree(dst3, zeros_t, ones_c)

    t, dinv = _pre(x_pad, W1, degp[0], degp[1])

    p = _sc_propagate(t, src3, dst3, zeros_t)
    t = _boundary1(p, t, dinv, b1)

    for W, b in ((W2, b2), (W3, b3), (W4, b4), (W5, b5)):
        p = _sc_propagate(t, src3, dst3, zeros_t)
        t = _boundary(p, t, dinv, W, b)

    p = _sc_propagate(t, src3, dst3, zeros_t)
    out = _final(p, t, dinv, W6, b6, W6.shape[1])
    return out[:N]


def kernel(x, edge_index, W1, b1, W2, b2, W3, b3, W4, b4, W5, b5, W6, b6):
    return _run(x, edge_index, W1, b1, W2, b2, W3, b3, W4, b4, W5, b5, W6, b6)


# speedup vs baseline: 30.5002x; 1.0152x over previous
import functools

import jax
import jax.numpy as jnp
from jax import lax
from jax.experimental import pallas as pl
from jax.experimental.pallas import tpu as pltpu
from jax.experimental.pallas import tpu_sc as plsc

N = 10000
E = 320000
D = 16
NPAD = 10240
NC, NS = 2, 16
NW = NC * NS
CH = 512
EPT = 10240
NCHUNK = EPT // CH
EPAD = NW * EPT
ROWS_PER_TILE = NPAD // NS


def _sc_propagate_body(table, src3, dst3, zeros, out, src_v, dst_v, rows_v,
                       sems, acc):
    c = lax.axis_index("c")
    s = lax.axis_index("s")

    pltpu.sync_copy(zeros.at[pl.ds(s * ROWS_PER_TILE, ROWS_PER_TILE)],
                    acc.at[pl.ds(s * ROWS_PER_TILE, ROWS_PER_TILE)])

    w = c * NS + s
    pltpu.sync_copy(src3.at[w], src_v)
    pltpu.sync_copy(dst3.at[w], dst_v)

    plsc.subcore_barrier()

    pltpu.async_copy(table.at[src_v.at[0]], rows_v.at[0], sems.at[0])

    def chunk(j, carry):
        nxt = j + 1

        @pl.when(nxt < NCHUNK)
        def _():
            pltpu.async_copy(table.at[src_v.at[nxt]], rows_v.at[nxt % 2],
                             sems.at[nxt % 2])

        pltpu.make_async_copy(table.at[src_v.at[j]], rows_v.at[j % 2],
                              sems.at[j % 2]).wait()
        pltpu.sync_copy(rows_v.at[j % 2], acc.at[dst_v.at[j]], add=True)
        return carry

    lax.fori_loop(0, NCHUNK, chunk, 0)

    plsc.subcore_barrier()

    pltpu.sync_copy(acc.at[pl.ds(s * ROWS_PER_TILE, ROWS_PER_TILE)],
                    out.at[c, pl.ds(s * ROWS_PER_TILE, ROWS_PER_TILE)])


_sc_propagate = pl.kernel(
    _sc_propagate_body,
    out_type=jax.ShapeDtypeStruct((NC, NPAD, D), jnp.float32),
    mesh=plsc.VectorSubcoreMesh(core_axis_name="c", subcore_axis_name="s"),
    scratch_types=[
        pltpu.VMEM((NCHUNK, CH), jnp.int32),
        pltpu.VMEM((NCHUNK, CH), jnp.int32),
        pltpu.VMEM((2, CH, D), jnp.float32),
        pltpu.SemaphoreType.DMA((2,)),
        pltpu.VMEM_SHARED((NPAD, D), jnp.float32),
    ],
    compiler_params=pltpu.CompilerParams(use_tc_tiling_on_sc=False),
)


def _sc_degree_body(dst3, zeros, ones16, out, dst_v, ones_v, acc):
    c = lax.axis_index("c")
    s = lax.axis_index("s")

    pltpu.sync_copy(zeros.at[pl.ds(s * ROWS_PER_TILE, ROWS_PER_TILE)],
                    acc.at[pl.ds(s * ROWS_PER_TILE, ROWS_PER_TILE)])

    w = c * NS + s
    pltpu.sync_copy(dst3.at[w], dst_v)
    pltpu.sync_copy(ones16, ones_v)

    plsc.subcore_barrier()

    def chunk(j, carry):
        pltpu.sync_copy(ones_v, acc.at[dst_v.at[j]], add=True)
        return carry

    lax.fori_loop(0, NCHUNK, chunk, 0)

    plsc.subcore_barrier()

    pltpu.sync_copy(acc.at[pl.ds(s * ROWS_PER_TILE, ROWS_PER_TILE)],
                    out.at[c, pl.ds(s * ROWS_PER_TILE, ROWS_PER_TILE)])


_sc_degree = pl.kernel(
    _sc_degree_body,
    out_type=jax.ShapeDtypeStruct((NC, NPAD, D), jnp.float32),
    mesh=plsc.VectorSubcoreMesh(core_axis_name="c", subcore_axis_name="s"),
    scratch_types=[
        pltpu.VMEM((NCHUNK, CH), jnp.int32),
        pltpu.VMEM((CH, D), jnp.float32),
        pltpu.VMEM_SHARED((NPAD, D), jnp.float32),
    ],
    compiler_params=pltpu.CompilerParams(use_tc_tiling_on_sc=False),
)


_RB = 1280
_GRID = NPAD // _RB


def _pre_body(x_ref, w_ref, d0_ref, d1_ref, t_ref, dinv_ref):
    deg = d0_ref[...] + d1_ref[...] + 1.0
    dinv = lax.rsqrt(deg)
    h = jnp.dot(x_ref[...], w_ref[...], preferred_element_type=jnp.float32)
    dinv_ref[...] = dinv
    t_ref[...] = h * dinv


def _pre(x, W1, d0, d1):
    return pl.pallas_call(
        _pre_body,
        grid=(_GRID,),
        in_specs=[
            pl.BlockSpec((_RB, 128), lambda i: (i, 0)),
            pl.BlockSpec((128, D), lambda i: (0, 0)),
            pl.BlockSpec((_RB, D), lambda i: (i, 0)),
            pl.BlockSpec((_RB, D), lambda i: (i, 0)),
        ],
        out_specs=[
            pl.BlockSpec((_RB, D), lambda i: (i, 0)),
            pl.BlockSpec((_RB, D), lambda i: (i, 0)),
        ],
        out_shape=[
            jax.ShapeDtypeStruct((NPAD, D), jnp.float32),
            jax.ShapeDtypeStruct((NPAD, D), jnp.float32),
        ],
    )(x, W1, d0, d1)


def _bnd1_body(p_ref, t_ref, dinv_ref, b_ref, o_ref):
    q = dinv_ref[...] * (p_ref[0] + p_ref[1] + t_ref[...])
    a = jnp.maximum(q + b_ref[...], 0.0)
    o_ref[...] = dinv_ref[...] * a


def _boundary1(p, t, dinv, b):
    return pl.pallas_call(
        _bnd1_body,
        grid=(_GRID,),
        in_specs=[
            pl.BlockSpec((NC, _RB, D), lambda i: (0, i, 0)),
            pl.BlockSpec((_RB, D), lambda i: (i, 0)),
            pl.BlockSpec((_RB, D), lambda i: (i, 0)),
            pl.BlockSpec((1, D), lambda i: (0, 0)),
        ],
        out_specs=pl.BlockSpec((_RB, D), lambda i: (i, 0)),
        out_shape=jax.ShapeDtypeStruct((NPAD, D), jnp.float32),
    )(p, t, dinv, b.reshape(1, D))


def _bnd_body(p_ref, t_ref, dinv_ref, w_ref, b_ref, o_ref):
    q = dinv_ref[...] * (p_ref[0] + p_ref[1] + t_ref[...])
    z = jnp.dot(q, w_ref[...], preferred_element_type=jnp.float32)
    a = jnp.maximum(z + b_ref[...], 0.0)
    o_ref[...] = dinv_ref[...] * a


def _boundary(p, t, dinv, W, b):
    return pl.pallas_call(
        _bnd_body,
        grid=(_GRID,),
        in_specs=[
            pl.BlockSpec((NC, _RB, D), lambda i: (0, i, 0)),
            pl.BlockSpec((_RB, D), lambda i: (i, 0)),
            pl.BlockSpec((_RB, D), lambda i: (i, 0)),
            pl.BlockSpec((D, D), lambda i: (0, 0)),
            pl.BlockSpec((1, D), lambda i: (0, 0)),
        ],
        out_specs=pl.BlockSpec((_RB, D), lambda i: (i, 0)),
        out_shape=jax.ShapeDtypeStruct((NPAD, D), jnp.float32),
    )(p, t, dinv, W, b.reshape(1, D))


def _final_body(p_ref, t_ref, dinv_ref, w_ref, b_ref, o_ref):
    q = dinv_ref[...] * (p_ref[0] + p_ref[1] + t_ref[...])
    z = jnp.dot(q, w_ref[...], preferred_element_type=jnp.float32)
    z = z + b_ref[...]
    m = jnp.max(z, axis=1, keepdims=True)
    zs = z - m
    lse = jnp.log(jnp.sum(jnp.exp(zs), axis=1, keepdims=True))
    o_ref[...] = zs - lse


def _final(p, t, dinv, W, b, dout):
    return pl.pallas_call(
        _final_body,
        grid=(_GRID,),
        in_specs=[
            pl.BlockSpec((NC, _RB, D), lambda i: (0, i, 0)),
            pl.BlockSpec((_RB, D), lambda i: (i, 0)),
            pl.BlockSpec((_RB, D), lambda i: (i, 0)),
            pl.BlockSpec((D, dout), lambda i: (0, 0)),
            pl.BlockSpec((1, dout), lambda i: (0, 0)),
        ],
        out_specs=pl.BlockSpec((_RB, dout), lambda i: (i, 0)),
        out_shape=jax.ShapeDtypeStruct((NPAD, dout), jnp.float32),
    )(p, t, dinv, W, b.reshape(1, dout))


@jax.jit
def _run(x, edge_index, W1, b1, W2, b2, W3, b3, W4, b4, W5, b5, W6, b6):
    src = edge_index[0].astype(jnp.int32)
    dst = edge_index[1].astype(jnp.int32)
    fill = jnp.full((EPAD - E,), NPAD - 1, dtype=jnp.int32)
    src3 = jnp.concatenate([src, fill]).reshape(NW, NCHUNK, CH)
    dst3 = jnp.concatenate([dst, fill]).reshape(NW, NCHUNK, CH)

    zeros_t = jnp.zeros((NPAD, D), jnp.float32)
    ones_c = jnp.ones((CH, D), jnp.float32)
    x_pad = jnp.concatenate([x, jnp.zeros((NPAD - N, 128), jnp.float32)])

    degp = _sc_degree(dst3, zeros_t, ones_c)

    t, dinv = _pre(x_pad, W1, degp[0], degp[1])

    p = _sc_propagate(t, src3, dst3, zeros_t)
    t = _boundary1(p, t, dinv, b1)

    for W, b in ((W2, b2), (W3, b3), (W4, b4), (W5, b5)):
        p = _sc_propagate(t, src3, dst3, zeros_t)
        t = _boundary(p, t, dinv, W, b)

    p = _sc_propagate(t, src3, dst3, zeros_t)
    out = _final(p, t, dinv, W6, b6, W6.shape[1])
    return out[:N]


def kernel(x, edge_index, W1, b1, W2, b2, W3, b3, W4, b4, W5, b5, W6, b6):
    return _run(x, edge_index, W1, b1, W2, b2, W3, b3, W4, b4, W5, b5, W6, b6)


# speedup vs baseline: 43.9391x; 1.4406x over previous
import functools

import jax
import jax.numpy as jnp
from jax import lax
from jax.experimental import pallas as pl
from jax.experimental.pallas import tpu as pltpu
from jax.experimental.pallas import tpu_sc as plsc

N = 10000
E = 320000
D = 16
NPAD = 10240
NC, NS = 2, 16
NW = NC * NS
CH = 512
EPT = 10240
NCHUNK = EPT // CH
EPAD = NW * EPT
ROWS_PER_TILE = NPAD // NS


def _sc_propagate_body(table, src3, dst3, zeros, out, src_v, dst_v, rows_v,
                       sems, acc, table_s):
    c = lax.axis_index("c")
    s = lax.axis_index("s")

    pltpu.sync_copy(zeros.at[pl.ds(s * ROWS_PER_TILE, ROWS_PER_TILE)],
                    acc.at[pl.ds(s * ROWS_PER_TILE, ROWS_PER_TILE)])
    pltpu.sync_copy(table.at[pl.ds(s * ROWS_PER_TILE, ROWS_PER_TILE)],
                    table_s.at[pl.ds(s * ROWS_PER_TILE, ROWS_PER_TILE)])

    w = c * NS + s
    pltpu.sync_copy(src3.at[w], src_v)
    pltpu.sync_copy(dst3.at[w], dst_v)

    plsc.subcore_barrier()

    pltpu.async_copy(table_s.at[src_v.at[0]], rows_v.at[0], sems.at[0])

    def chunk(j, carry):
        nxt = j + 1

        @pl.when(nxt < NCHUNK)
        def _():
            pltpu.async_copy(table_s.at[src_v.at[nxt]], rows_v.at[nxt % 2],
                             sems.at[nxt % 2])

        pltpu.make_async_copy(table_s.at[src_v.at[j]], rows_v.at[j % 2],
                              sems.at[j % 2]).wait()
        pltpu.sync_copy(rows_v.at[j % 2], acc.at[dst_v.at[j]], add=True)
        return carry

    lax.fori_loop(0, NCHUNK, chunk, 0)

    plsc.subcore_barrier()

    pltpu.sync_copy(acc.at[pl.ds(s * ROWS_PER_TILE, ROWS_PER_TILE)],
                    out.at[c, pl.ds(s * ROWS_PER_TILE, ROWS_PER_TILE)])


_sc_propagate = pl.kernel(
    _sc_propagate_body,
    out_type=jax.ShapeDtypeStruct((NC, NPAD, D), jnp.float32),
    mesh=plsc.VectorSubcoreMesh(core_axis_name="c", subcore_axis_name="s"),
    scratch_types=[
        pltpu.VMEM((NCHUNK, CH), jnp.int32),
        pltpu.VMEM((NCHUNK, CH), jnp.int32),
        pltpu.VMEM((2, CH, D), jnp.float32),
        pltpu.SemaphoreType.DMA((2,)),
        pltpu.VMEM_SHARED((NPAD, D), jnp.float32),
        pltpu.VMEM_SHARED((NPAD, D), jnp.float32),
    ],
    compiler_params=pltpu.CompilerParams(use_tc_tiling_on_sc=False),
)


def _sc_degree_body(dst3, zeros, ones16, out, dst_v, ones_v, acc):
    c = lax.axis_index("c")
    s = lax.axis_index("s")

    pltpu.sync_copy(zeros.at[pl.ds(s * ROWS_PER_TILE, ROWS_PER_TILE)],
                    acc.at[pl.ds(s * ROWS_PER_TILE, ROWS_PER_TILE)])

    w = c * NS + s
    pltpu.sync_copy(dst3.at[w], dst_v)
    pltpu.sync_copy(ones16, ones_v)

    plsc.subcore_barrier()

    def chunk(j, carry):
        pltpu.sync_copy(ones_v, acc.at[dst_v.at[j]], add=True)
        return carry

    lax.fori_loop(0, NCHUNK, chunk, 0)

    plsc.subcore_barrier()

    pltpu.sync_copy(acc.at[pl.ds(s * ROWS_PER_TILE, ROWS_PER_TILE)],
                    out.at[c, pl.ds(s * ROWS_PER_TILE, ROWS_PER_TILE)])


_sc_degree = pl.kernel(
    _sc_degree_body,
    out_type=jax.ShapeDtypeStruct((NC, NPAD, D), jnp.float32),
    mesh=plsc.VectorSubcoreMesh(core_axis_name="c", subcore_axis_name="s"),
    scratch_types=[
        pltpu.VMEM((NCHUNK, CH), jnp.int32),
        pltpu.VMEM((CH, D), jnp.float32),
        pltpu.VMEM_SHARED((NPAD, D), jnp.float32),
    ],
    compiler_params=pltpu.CompilerParams(use_tc_tiling_on_sc=False),
)


_RB = 1280
_GRID = NPAD // _RB


def _pre_body(x_ref, w_ref, d0_ref, d1_ref, t_ref, dinv_ref):
    deg = d0_ref[...] + d1_ref[...] + 1.0
    dinv = lax.rsqrt(deg)
    h = jnp.dot(x_ref[...], w_ref[...], preferred_element_type=jnp.float32)
    dinv_ref[...] = dinv
    t_ref[...] = h * dinv


def _pre(x, W1, d0, d1):
    return pl.pallas_call(
        _pre_body,
        grid=(_GRID,),
        in_specs=[
            pl.BlockSpec((_RB, 128), lambda i: (i, 0)),
            pl.BlockSpec((128, D), lambda i: (0, 0)),
            pl.BlockSpec((_RB, D), lambda i: (i, 0)),
            pl.BlockSpec((_RB, D), lambda i: (i, 0)),
        ],
        out_specs=[
            pl.BlockSpec((_RB, D), lambda i: (i, 0)),
            pl.BlockSpec((_RB, D), lambda i: (i, 0)),
        ],
        out_shape=[
            jax.ShapeDtypeStruct((NPAD, D), jnp.float32),
            jax.ShapeDtypeStruct((NPAD, D), jnp.float32),
        ],
    )(x, W1, d0, d1)


def _bnd1_body(p_ref, t_ref, dinv_ref, b_ref, o_ref):
    q = dinv_ref[...] * (p_ref[0] + p_ref[1] + t_ref[...])
    a = jnp.maximum(q + b_ref[...], 0.0)
    o_ref[...] = dinv_ref[...] * a


def _boundary1(p, t, dinv, b):
    return pl.pallas_call(
        _bnd1_body,
        grid=(_GRID,),
        in_specs=[
            pl.BlockSpec((NC, _RB, D), lambda i: (0, i, 0)),
            pl.BlockSpec((_RB, D), lambda i: (i, 0)),
            pl.BlockSpec((_RB, D), lambda i: (i, 0)),
            pl.BlockSpec((1, D), lambda i: (0, 0)),
        ],
        out_specs=pl.BlockSpec((_RB, D), lambda i: (i, 0)),
        out_shape=jax.ShapeDtypeStruct((NPAD, D), jnp.float32),
    )(p, t, dinv, b.reshape(1, D))


def _bnd_body(p_ref, t_ref, dinv_ref, w_ref, b_ref, o_ref):
    q = dinv_ref[...] * (p_ref[0] + p_ref[1] + t_ref[...])
    z = jnp.dot(q, w_ref[...], preferred_element_type=jnp.float32)
    a = jnp.maximum(z + b_ref[...], 0.0)
    o_ref[...] = dinv_ref[...] * a


def _boundary(p, t, dinv, W, b):
    return pl.pallas_call(
        _bnd_body,
        grid=(_GRID,),
        in_specs=[
            pl.BlockSpec((NC, _RB, D), lambda i: (0, i, 0)),
            pl.BlockSpec((_RB, D), lambda i: (i, 0)),
            pl.BlockSpec((_RB, D), lambda i: (i, 0)),
            pl.BlockSpec((D, D), lambda i: (0, 0)),
            pl.BlockSpec((1, D), lambda i: (0, 0)),
        ],
        out_specs=pl.BlockSpec((_RB, D), lambda i: (i, 0)),
        out_shape=jax.ShapeDtypeStruct((NPAD, D), jnp.float32),
    )(p, t, dinv, W, b.reshape(1, D))


def _final_body(p_ref, t_ref, dinv_ref, w_ref, b_ref, o_ref):
    q = dinv_ref[...] * (p_ref[0] + p_ref[1] + t_ref[...])
    z = jnp.dot(q, w_ref[...], preferred_element_type=jnp.float32)
    z = z + b_ref[...]
    m = jnp.max(z, axis=1, keepdims=True)
    zs = z - m
    lse = jnp.log(jnp.sum(jnp.exp(zs), axis=1, keepdims=True))
    o_ref[...] = zs - lse


def _final(p, t, dinv, W, b, dout):
    return pl.pallas_call(
        _final_body,
        grid=(_GRID,),
        in_specs=[
            pl.BlockSpec((NC, _RB, D), lambda i: (0, i, 0)),
            pl.BlockSpec((_RB, D), lambda i: (i, 0)),
            pl.BlockSpec((_RB, D), lambda i: (i, 0)),
            pl.BlockSpec((D, dout), lambda i: (0, 0)),
            pl.BlockSpec((1, dout), lambda i: (0, 0)),
        ],
        out_specs=pl.BlockSpec((_RB, dout), lambda i: (i, 0)),
        out_shape=jax.ShapeDtypeStruct((NPAD, dout), jnp.float32),
    )(p, t, dinv, W, b.reshape(1, dout))


@jax.jit
def _run(x, edge_index, W1, b1, W2, b2, W3, b3, W4, b4, W5, b5, W6, b6):
    src = edge_index[0].astype(jnp.int32)
    dst = edge_index[1].astype(jnp.int32)
    fill = jnp.full((EPAD - E,), NPAD - 1, dtype=jnp.int32)
    src3 = jnp.concatenate([src, fill]).reshape(NW, NCHUNK, CH)
    dst3 = jnp.concatenate([dst, fill]).reshape(NW, NCHUNK, CH)

    zeros_t = jnp.zeros((NPAD, D), jnp.float32)
    ones_c = jnp.ones((CH, D), jnp.float32)
    x_pad = jnp.concatenate([x, jnp.zeros((NPAD - N, 128), jnp.float32)])

    degp = _sc_degree(dst3, zeros_t, ones_c)

    t, dinv = _pre(x_pad, W1, degp[0], degp[1])

    p = _sc_propagate(t, src3, dst3, zeros_t)
    t = _boundary1(p, t, dinv, b1)

    for W, b in ((W2, b2), (W3, b3), (W4, b4), (W5, b5)):
        p = _sc_propagate(t, src3, dst3, zeros_t)
        t = _boundary(p, t, dinv, W, b)

    p = _sc_propagate(t, src3, dst3, zeros_t)
    out = _final(p, t, dinv, W6, b6, W6.shape[1])
    return out[:N]


def kernel(x, edge_index, W1, b1, W2, b2, W3, b3, W4, b4, W5, b5, W6, b6):
    return _run(x, edge_index, W1, b1, W2, b2, W3, b3, W4, b4, W5, b5, W6, b6)


# speedup vs baseline: 57.5725x; 1.3103x over previous
import functools

import jax
import jax.numpy as jnp
from jax import lax
from jax.experimental import pallas as pl
from jax.experimental.pallas import tpu as pltpu
from jax.experimental.pallas import tpu_sc as plsc

N = 10000
E = 320000
D = 16
NPAD = 10240
NC, NS = 2, 16
NW = NC * NS
CH = 512
EPT = 10240
NCHUNK = EPT // CH
EPAD = NW * EPT
ROWS_PER_TILE = NPAD // NS


def _sc_propagate_body(table, src3, dst3, zeros, out, src_v, dst_v, rows_v,
                       sems, acc, table_s):
    c = lax.axis_index("c")
    s = lax.axis_index("s")

    pltpu.sync_copy(zeros.at[pl.ds(s * ROWS_PER_TILE, ROWS_PER_TILE)],
                    acc.at[pl.ds(s * ROWS_PER_TILE, ROWS_PER_TILE)])
    pltpu.sync_copy(table.at[pl.ds(s * ROWS_PER_TILE, ROWS_PER_TILE)],
                    table_s.at[pl.ds(s * ROWS_PER_TILE, ROWS_PER_TILE)])

    w = c * NS + s
    pltpu.sync_copy(src3.at[w], src_v)
    pltpu.sync_copy(dst3.at[w], dst_v)

    plsc.subcore_barrier()

    pltpu.async_copy(table_s.at[src_v.at[0]], rows_v.at[0], sems.at[0])

    def chunk(j, carry):
        nxt = j + 1

        @pl.when(nxt < NCHUNK)
        def _():
            pltpu.async_copy(table_s.at[src_v.at[nxt]], rows_v.at[nxt % 2],
                             sems.at[nxt % 2])

        pltpu.make_async_copy(table_s.at[src_v.at[j]], rows_v.at[j % 2],
                              sems.at[j % 2]).wait()
        pltpu.sync_copy(rows_v.at[j % 2], acc.at[dst_v.at[j]], add=True)
        return carry

    lax.fori_loop(0, NCHUNK, chunk, 0)

    plsc.subcore_barrier()

    pltpu.sync_copy(acc.at[pl.ds(s * ROWS_PER_TILE, ROWS_PER_TILE)],
                    out.at[c, pl.ds(s * ROWS_PER_TILE, ROWS_PER_TILE)])


_sc_propagate = pl.kernel(
    _sc_propagate_body,
    out_type=jax.ShapeDtypeStruct((NC, NPAD, D), jnp.float32),
    mesh=plsc.VectorSubcoreMesh(core_axis_name="c", subcore_axis_name="s"),
    scratch_types=[
        pltpu.VMEM((NCHUNK, CH), jnp.int32),
        pltpu.VMEM((NCHUNK, CH), jnp.int32),
        pltpu.VMEM((2, CH, D), jnp.float32),
        pltpu.SemaphoreType.DMA((2,)),
        pltpu.VMEM_SHARED((NPAD, D), jnp.float32),
        pltpu.VMEM_SHARED((NPAD, D), jnp.float32),
    ],
    compiler_params=pltpu.CompilerParams(use_tc_tiling_on_sc=False),
)


def _sc_degree_body(dst3, zeros, ones16, out, dst_v, ones_v, acc):
    c = lax.axis_index("c")
    s = lax.axis_index("s")

    pltpu.sync_copy(zeros.at[pl.ds(s * ROWS_PER_TILE, ROWS_PER_TILE)],
                    acc.at[pl.ds(s * ROWS_PER_TILE, ROWS_PER_TILE)])

    w = c * NS + s
    pltpu.sync_copy(dst3.at[w], dst_v)
    pltpu.sync_copy(ones16, ones_v)

    plsc.subcore_barrier()

    def chunk(j, carry):
        pltpu.sync_copy(ones_v, acc.at[dst_v.at[j]], add=True)
        return carry

    lax.fori_loop(0, NCHUNK, chunk, 0)

    plsc.subcore_barrier()

    pltpu.sync_copy(acc.at[pl.ds(s * ROWS_PER_TILE, ROWS_PER_TILE)],
                    out.at[c, pl.ds(s * ROWS_PER_TILE, ROWS_PER_TILE)])


_sc_degree = pl.kernel(
    _sc_degree_body,
    out_type=jax.ShapeDtypeStruct((NC, NPAD, D), jnp.float32),
    mesh=plsc.VectorSubcoreMesh(core_axis_name="c", subcore_axis_name="s"),
    scratch_types=[
        pltpu.VMEM((NCHUNK, CH), jnp.int32),
        pltpu.VMEM((CH, D), jnp.float32),
        pltpu.VMEM_SHARED((NPAD, D), jnp.float32),
    ],
    compiler_params=pltpu.CompilerParams(use_tc_tiling_on_sc=False),
)


NP8 = NPAD * D // 128
_RB = 160
_GRID = NP8 // _RB


def _pre_body(x_ref, w_ref, d0_ref, d1_ref, t_ref, dinv_ref):
    deg = d0_ref[...] + d1_ref[...] + 1.0
    dinv = lax.rsqrt(deg)
    h = jnp.dot(x_ref[...], w_ref[...], preferred_element_type=jnp.float32)
    dinv_ref[...] = dinv
    t_ref[...] = h * dinv


def _pre(x8, W1_8, d0, d1):
    return pl.pallas_call(
        _pre_body,
        grid=(_GRID,),
        in_specs=[
            pl.BlockSpec((_RB, 1024), lambda i: (i, 0)),
            pl.BlockSpec((1024, 128), lambda i: (0, 0)),
            pl.BlockSpec((_RB, 128), lambda i: (i, 0)),
            pl.BlockSpec((_RB, 128), lambda i: (i, 0)),
        ],
        out_specs=[
            pl.BlockSpec((_RB, 128), lambda i: (i, 0)),
            pl.BlockSpec((_RB, 128), lambda i: (i, 0)),
        ],
        out_shape=[
            jax.ShapeDtypeStruct((NP8, 128), jnp.float32),
            jax.ShapeDtypeStruct((NP8, 128), jnp.float32),
        ],
    )(x8, W1_8, d0, d1)


def _bnd1_body(p_ref, t_ref, dinv_ref, b_ref, o_ref):
    q = dinv_ref[...] * (p_ref[0] + p_ref[1] + t_ref[...])
    a = jnp.maximum(q + b_ref[...], 0.0)
    o_ref[...] = dinv_ref[...] * a


def _boundary1(p, t, dinv, b8):
    return pl.pallas_call(
        _bnd1_body,
        grid=(_GRID,),
        in_specs=[
            pl.BlockSpec((NC, _RB, 128), lambda i: (0, i, 0)),
            pl.BlockSpec((_RB, 128), lambda i: (i, 0)),
            pl.BlockSpec((_RB, 128), lambda i: (i, 0)),
            pl.BlockSpec((1, 128), lambda i: (0, 0)),
        ],
        out_specs=pl.BlockSpec((_RB, 128), lambda i: (i, 0)),
        out_shape=jax.ShapeDtypeStruct((NP8, 128), jnp.float32),
    )(p, t, dinv, b8)


def _bnd_body(p_ref, t_ref, dinv_ref, w_ref, b_ref, o_ref):
    q = dinv_ref[...] * (p_ref[0] + p_ref[1] + t_ref[...])
    z = jnp.dot(q, w_ref[...], preferred_element_type=jnp.float32)
    a = jnp.maximum(z + b_ref[...], 0.0)
    o_ref[...] = dinv_ref[...] * a


def _boundary(p, t, dinv, W8, b8):
    return pl.pallas_call(
        _bnd_body,
        grid=(_GRID,),
        in_specs=[
            pl.BlockSpec((NC, _RB, 128), lambda i: (0, i, 0)),
            pl.BlockSpec((_RB, 128), lambda i: (i, 0)),
            pl.BlockSpec((_RB, 128), lambda i: (i, 0)),
            pl.BlockSpec((128, 128), lambda i: (0, 0)),
            pl.BlockSpec((1, 128), lambda i: (0, 0)),
        ],
        out_specs=pl.BlockSpec((_RB, 128), lambda i: (i, 0)),
        out_shape=jax.ShapeDtypeStruct((NP8, 128), jnp.float32),
    )(p, t, dinv, W8, b8)


def _fmm_body(p_ref, t_ref, dinv_ref, w_ref, b_ref, o_ref):
    q = dinv_ref[...] * (p_ref[0] + p_ref[1] + t_ref[...])
    z = jnp.dot(q, w_ref[...], preferred_element_type=jnp.float32)
    o_ref[...] = z + b_ref[...]


def _final_matmul(p, t, dinv, W6_8, b6_8):
    return pl.pallas_call(
        _fmm_body,
        grid=(_GRID,),
        in_specs=[
            pl.BlockSpec((NC, _RB, 128), lambda i: (0, i, 0)),
            pl.BlockSpec((_RB, 128), lambda i: (i, 0)),
            pl.BlockSpec((_RB, 128), lambda i: (i, 0)),
            pl.BlockSpec((128, 512), lambda i: (0, 0)),
            pl.BlockSpec((1, 512), lambda i: (0, 0)),
        ],
        out_specs=pl.BlockSpec((_RB, 512), lambda i: (i, 0)),
        out_shape=jax.ShapeDtypeStruct((NP8, 512), jnp.float32),
    )(p, t, dinv, W6_8, b6_8)


def _lsm_body(z_ref, o_ref):
    z = z_ref[...]
    m = jnp.max(z, axis=1, keepdims=True)
    zs = z - m
    lse = jnp.log(jnp.sum(jnp.exp(zs), axis=1, keepdims=True))
    o_ref[...] = zs - lse


def _log_softmax(z):
    return pl.pallas_call(
        _lsm_body,
        grid=(_GRID,),
        in_specs=[pl.BlockSpec((NPAD // _GRID, 64), lambda i: (i, 0))],
        out_specs=pl.BlockSpec((NPAD // _GRID, 64), lambda i: (i, 0)),
        out_shape=jax.ShapeDtypeStruct((NPAD, 64), jnp.float32),
    )(z)


@jax.jit
def _run(x, edge_index, W1, b1, W2, b2, W3, b3, W4, b4, W5, b5, W6, b6):
    src = edge_index[0].astype(jnp.int32)
    dst = edge_index[1].astype(jnp.int32)
    fill = jnp.full((EPAD - E,), NPAD - 1, dtype=jnp.int32)
    src3 = jnp.concatenate([src, fill]).reshape(NW, NCHUNK, CH)
    dst3 = jnp.concatenate([dst, fill]).reshape(NW, NCHUNK, CH)

    zeros_t = jnp.zeros((NPAD, D), jnp.float32)
    ones_c = jnp.ones((CH, D), jnp.float32)
    x_pad = jnp.concatenate([x, jnp.zeros((NPAD - N, 128), jnp.float32)])
    x8 = x_pad.reshape(NP8, 1024)

    eye8 = jnp.eye(8, dtype=jnp.float32)
    W1_8 = jnp.kron(eye8, W1)
    W6_8 = jnp.kron(eye8, W6)
    b1_8 = jnp.tile(b1, 8).reshape(1, 128)
    b6_8 = jnp.tile(b6, 8).reshape(1, 512)

    def packed(a):
        return a.reshape(-1, NP8, 128) if a.ndim == 3 else a.reshape(NP8, 128)

    degp = packed(_sc_degree(dst3, zeros_t, ones_c))

    t, dinv = _pre(x8, W1_8, degp[0], degp[1])

    p = packed(_sc_propagate(t.reshape(NPAD, D), src3, dst3, zeros_t))
    t = _boundary1(p, t, dinv, b1_8)

    for W, b in ((W2, b2), (W3, b3), (W4, b4), (W5, b5)):
        p = packed(_sc_propagate(t.reshape(NPAD, D), src3, dst3, zeros_t))
        t = _boundary(p, t, dinv, jnp.kron(eye8, W), jnp.tile(b, 8).reshape(1, 128))

    p = packed(_sc_propagate(t.reshape(NPAD, D), src3, dst3, zeros_t))
    z = _final_matmul(p, t, dinv, W6_8, b6_8)
    out = _log_softmax(z.reshape(NPAD, 64))
    return out[:N]


def kernel(x, edge_index, W1, b1, W2, b2, W3, b3, W4, b4, W5, b5, W6, b6):
    return _run(x, edge_index, W1, b1, W2, b2, W3, b3, W4, b4, W5, b5, W6, b6)


# speedup vs baseline: 66.7411x; 1.1593x over previous
import functools

import jax
import jax.numpy as jnp
from jax import lax
from jax.experimental import pallas as pl
from jax.experimental.pallas import tpu as pltpu
from jax.experimental.pallas import tpu_sc as plsc

N = 10000
E = 320000
D = 16
NPAD = 10240
NC, NS = 2, 16
NW = NC * NS
CH = 500
EPT = E // NW
NCHUNK = EPT // CH
ROWS_PER_TILE = NPAD // NS


def _sc_propagate_body(table, src3, dst3, zeros, out, src_v, dst_v, rows_v,
                       sems, acc, table_s):
    c = lax.axis_index("c")
    s = lax.axis_index("s")

    pltpu.sync_copy(zeros.at[pl.ds(s * ROWS_PER_TILE, ROWS_PER_TILE)],
                    acc.at[pl.ds(s * ROWS_PER_TILE, ROWS_PER_TILE)])
    pltpu.sync_copy(table.at[pl.ds(s * ROWS_PER_TILE, ROWS_PER_TILE)],
                    table_s.at[pl.ds(s * ROWS_PER_TILE, ROWS_PER_TILE)])

    w = c * NS + s
    pltpu.sync_copy(src3.at[w], src_v)
    pltpu.sync_copy(dst3.at[w], dst_v)

    plsc.subcore_barrier()

    pltpu.async_copy(table_s.at[src_v.at[0]], rows_v.at[0], sems.at[0])

    def chunk(j, carry):
        nxt = j + 1

        @pl.when(nxt < NCHUNK)
        def _():
            pltpu.async_copy(table_s.at[src_v.at[nxt]], rows_v.at[nxt % 2],
                             sems.at[nxt % 2])

        pltpu.make_async_copy(table_s.at[src_v.at[j]], rows_v.at[j % 2],
                              sems.at[j % 2]).wait()
        pltpu.sync_copy(rows_v.at[j % 2], acc.at[dst_v.at[j]], add=True)
        return carry

    lax.fori_loop(0, NCHUNK, chunk, 0)

    plsc.subcore_barrier()

    pltpu.sync_copy(acc.at[pl.ds(s * ROWS_PER_TILE, ROWS_PER_TILE)],
                    out.at[c, pl.ds(s * ROWS_PER_TILE, ROWS_PER_TILE)])


_sc_propagate = pl.kernel(
    _sc_propagate_body,
    out_type=jax.ShapeDtypeStruct((NC, NPAD, D), jnp.float32),
    mesh=plsc.VectorSubcoreMesh(core_axis_name="c", subcore_axis_name="s"),
    scratch_types=[
        pltpu.VMEM((NCHUNK, CH), jnp.int32),
        pltpu.VMEM((NCHUNK, CH), jnp.int32),
        pltpu.VMEM((2, CH, D), jnp.float32),
        pltpu.SemaphoreType.DMA((2,)),
        pltpu.VMEM_SHARED((NPAD, D), jnp.float32),
        pltpu.VMEM_SHARED((NPAD, D), jnp.float32),
    ],
    compiler_params=pltpu.CompilerParams(use_tc_tiling_on_sc=False),
)


def _sc_degree_body(dst3, zeros, ones16, out, dst_v, ones_v, acc):
    c = lax.axis_index("c")
    s = lax.axis_index("s")

    pltpu.sync_copy(zeros.at[pl.ds(s * ROWS_PER_TILE, ROWS_PER_TILE)],
                    acc.at[pl.ds(s * ROWS_PER_TILE, ROWS_PER_TILE)])

    w = c * NS + s
    pltpu.sync_copy(dst3.at[w], dst_v)
    pltpu.sync_copy(ones16, ones_v)

    plsc.subcore_barrier()

    def chunk(j, carry):
        pltpu.sync_copy(ones_v, acc.at[dst_v.at[j]], add=True)
        return carry

    lax.fori_loop(0, NCHUNK, chunk, 0)

    plsc.subcore_barrier()

    pltpu.sync_copy(acc.at[pl.ds(s * ROWS_PER_TILE, ROWS_PER_TILE)],
                    out.at[c, pl.ds(s * ROWS_PER_TILE, ROWS_PER_TILE)])


_sc_degree = pl.kernel(
    _sc_degree_body,
    out_type=jax.ShapeDtypeStruct((NC, NPAD, D), jnp.float32),
    mesh=plsc.VectorSubcoreMesh(core_axis_name="c", subcore_axis_name="s"),
    scratch_types=[
        pltpu.VMEM((NCHUNK, CH), jnp.int32),
        pltpu.VMEM((CH, D), jnp.float32),
        pltpu.VMEM_SHARED((NPAD, D), jnp.float32),
    ],
    compiler_params=pltpu.CompilerParams(use_tc_tiling_on_sc=False),
)


NP8 = NPAD * D // 128
_RB = 160
_GRID = NP8 // _RB


def _pre_body(x_ref, w_ref, d0_ref, d1_ref, t_ref, dinv_ref):
    deg = d0_ref[...] + d1_ref[...] + 1.0
    dinv = lax.rsqrt(deg)
    h = jnp.dot(x_ref[...], w_ref[...], preferred_element_type=jnp.float32)
    dinv_ref[...] = dinv
    t_ref[...] = h * dinv


def _pre(x8, W1_8, d0, d1):
    return pl.pallas_call(
        _pre_body,
        grid=(_GRID,),
        in_specs=[
            pl.BlockSpec((_RB, 1024), lambda i: (i, 0)),
            pl.BlockSpec((1024, 128), lambda i: (0, 0)),
            pl.BlockSpec((_RB, 128), lambda i: (i, 0)),
            pl.BlockSpec((_RB, 128), lambda i: (i, 0)),
        ],
        out_specs=[
            pl.BlockSpec((_RB, 128), lambda i: (i, 0)),
            pl.BlockSpec((_RB, 128), lambda i: (i, 0)),
        ],
        out_shape=[
            jax.ShapeDtypeStruct((NP8, 128), jnp.float32),
            jax.ShapeDtypeStruct((NP8, 128), jnp.float32),
        ],
    )(x8, W1_8, d0, d1)


def _bnd1_body(p_ref, t_ref, dinv_ref, b_ref, o_ref):
    q = dinv_ref[...] * (p_ref[0] + p_ref[1] + t_ref[...])
    a = jnp.maximum(q + b_ref[...], 0.0)
    o_ref[...] = dinv_ref[...] * a


def _boundary1(p, t, dinv, b8):
    return pl.pallas_call(
        _bnd1_body,
        grid=(_GRID,),
        in_specs=[
            pl.BlockSpec((NC, _RB, 128), lambda i: (0, i, 0)),
            pl.BlockSpec((_RB, 128), lambda i: (i, 0)),
            pl.BlockSpec((_RB, 128), lambda i: (i, 0)),
            pl.BlockSpec((1, 128), lambda i: (0, 0)),
        ],
        out_specs=pl.BlockSpec((_RB, 128), lambda i: (i, 0)),
        out_shape=jax.ShapeDtypeStruct((NP8, 128), jnp.float32),
    )(p, t, dinv, b8)


def _bnd_body(p_ref, t_ref, dinv_ref, w_ref, b_ref, o_ref):
    q = dinv_ref[...] * (p_ref[0] + p_ref[1] + t_ref[...])
    z = jnp.dot(q, w_ref[...], preferred_element_type=jnp.float32)
    a = jnp.maximum(z + b_ref[...], 0.0)
    o_ref[...] = dinv_ref[...] * a


def _boundary(p, t, dinv, W8, b8):
    return pl.pallas_call(
        _bnd_body,
        grid=(_GRID,),
        in_specs=[
            pl.BlockSpec((NC, _RB, 128), lambda i: (0, i, 0)),
            pl.BlockSpec((_RB, 128), lambda i: (i, 0)),
            pl.BlockSpec((_RB, 128), lambda i: (i, 0)),
            pl.BlockSpec((128, 128), lambda i: (0, 0)),
            pl.BlockSpec((1, 128), lambda i: (0, 0)),
        ],
        out_specs=pl.BlockSpec((_RB, 128), lambda i: (i, 0)),
        out_shape=jax.ShapeDtypeStruct((NP8, 128), jnp.float32),
    )(p, t, dinv, W8, b8)


def _fmm_body(p_ref, t_ref, dinv_ref, w_ref, b_ref, o_ref):
    q = dinv_ref[...] * (p_ref[0] + p_ref[1] + t_ref[...])
    z = jnp.dot(q, w_ref[...], preferred_element_type=jnp.float32)
    o_ref[...] = z + b_ref[...]


def _final_matmul(p, t, dinv, W6_8, b6_8):
    return pl.pallas_call(
        _fmm_body,
        grid=(_GRID,),
        in_specs=[
            pl.BlockSpec((NC, _RB, 128), lambda i: (0, i, 0)),
            pl.BlockSpec((_RB, 128), lambda i: (i, 0)),
            pl.BlockSpec((_RB, 128), lambda i: (i, 0)),
            pl.BlockSpec((128, 512), lambda i: (0, 0)),
            pl.BlockSpec((1, 512), lambda i: (0, 0)),
        ],
        out_specs=pl.BlockSpec((_RB, 512), lambda i: (i, 0)),
        out_shape=jax.ShapeDtypeStruct((NP8, 512), jnp.float32),
    )(p, t, dinv, W6_8, b6_8)


def _lsm_body(z_ref, o_ref):
    z = z_ref[...]
    m = jnp.max(z, axis=1, keepdims=True)
    zs = z - m
    lse = jnp.log(jnp.sum(jnp.exp(zs), axis=1, keepdims=True))
    o_ref[...] = zs - lse


def _log_softmax(z):
    return pl.pallas_call(
        _lsm_body,
        grid=(_GRID,),
        in_specs=[pl.BlockSpec((NPAD // _GRID, 64), lambda i: (i, 0))],
        out_specs=pl.BlockSpec((NPAD // _GRID, 64), lambda i: (i, 0)),
        out_shape=jax.ShapeDtypeStruct((NPAD, 64), jnp.float32),
    )(z)


@jax.jit
def _run(x, edge_index, W1, b1, W2, b2, W3, b3, W4, b4, W5, b5, W6, b6):
    src3 = edge_index[0].astype(jnp.int32).reshape(NW, NCHUNK, CH)
    dst3 = edge_index[1].astype(jnp.int32).reshape(NW, NCHUNK, CH)

    zeros_t = jnp.zeros((NPAD, D), jnp.float32)
    ones_c = jnp.ones((CH, D), jnp.float32)
    x_pad = jnp.concatenate([x, jnp.zeros((NPAD - N, 128), jnp.float32)])
    x8 = x_pad.reshape(NP8, 1024)

    eye8 = jnp.eye(8, dtype=jnp.float32)
    W1_8 = jnp.kron(eye8, W1)
    W6_8 = jnp.kron(eye8, W6)
    b1_8 = jnp.tile(b1, 8).reshape(1, 128)
    b6_8 = jnp.tile(b6, 8).reshape(1, 512)

    def packed(a):
        return a.reshape(-1, NP8, 128) if a.ndim == 3 else a.reshape(NP8, 128)

    degp = packed(_sc_degree(dst3, zeros_t, ones_c))

    t, dinv = _pre(x8, W1_8, degp[0], degp[1])

    p = packed(_sc_propagate(t.reshape(NPAD, D), src3, dst3, zeros_t))
    t = _boundary1(p, t, dinv, b1_8)

    for W, b in ((W2, b2), (W3, b3), (W4, b4), (W5, b5)):
        p = packed(_sc_propagate(t.reshape(NPAD, D), src3, dst3, zeros_t))
        t = _boundary(p, t, dinv, jnp.kron(eye8, W), jnp.tile(b, 8).reshape(1, 128))

    p = packed(_sc_propagate(t.reshape(NPAD, D), src3, dst3, zeros_t))
    z = _final_matmul(p, t, dinv, W6_8, b6_8)
    out = _log_softmax(z.reshape(NPAD, 64))
    return out[:N]


def kernel(x, edge_index, W1, b1, W2, b2, W3, b3, W4, b4, W5, b5, W6, b6):
    return _run(x, edge_index, W1, b1, W2, b2, W3, b3, W4, b4, W5, b5, W6, b6)
